# baseline (speedup 1.0000x reference)
import functools

import jax
import jax.numpy as jnp
from jax import lax
from jax.experimental import pallas as pl
from jax.experimental.pallas import tpu as pltpu
from jax.experimental.pallas import tpu_sc as plsc

_N = 10000
_E = 320000
_NFEAT = 128
_NHID = 16
_NHEAD = 8
_NCLASS = 32
_ALPHA = 0.2

_NC = 2
_NS = 16
_C1 = 80
_C2 = 125

_GATHER_DNUMS = lax.GatherDimensionNumbers(
    offset_dims=(), collapsed_slice_dims=(0,), start_index_map=(0,))


def _bcast_lane(v, k):
    idx = jnp.full((16, 1), k, dtype=jnp.int32)
    return lax.gather(v, idx, _GATHER_DNUMS, (1,),
                      mode=lax.GatherScatterMode.PROMISE_IN_BOUNDS)


def _edge_pass(t_in, sdst_tab, src_rs, dst_rs, feat, nhead, C):
    row = feat + 16
    nblk = feat // 16
    iters = src_rs.shape[1]
    rpt = _N // _NS
    zc = 125 if C >= 125 else 25
    mesh = plsc.VectorSubcoreMesh(core_axis_name="c", subcore_axis_name="s",
                                  num_cores=_NC, num_subcores=_NS)
    idx_t = pltpu.VMEM((C,), jnp.int32)
    gat_t = pltpu.VMEM((C, feat // 2 + 16), jnp.int32)
    row_t = pltpu.VMEM((C, row), jnp.float32)
    sem_t = pltpu.SemaphoreType.DMA

    @functools.partial(
        pl.kernel,
        out_type=jax.ShapeDtypeStruct((_NC, _N, row), jnp.float32),
        mesh=mesh,
        scratch_types=(
            [pltpu.VMEM_SHARED((_N, row), jnp.float32)]
            + [idx_t] * 6
            + [gat_t, gat_t,
               pltpu.VMEM((C, 16), jnp.float32),
               pltpu.VMEM((C, 16), jnp.float32),
               row_t, row_t]
            + [sem_t] * 12
        ),
        compiler_params=pltpu.CompilerParams(use_tc_tiling_on_sc=False,
                                             needs_layout_passes=False),
    )
    def ek(t_in_hbm, sdst_hbm, src_hbm, dst_hbm, out_hbm,
           acc, sg0, sg1, dg0, dg1, ds0, ds1,
           ab0, ab1, bb0, bb1, ub0, ub1,
           ssg0, ssg1, sdg0, sdg1, sds0, sds1,
           sa0, sa1, sb0, sb1, su0, su1):
        c = lax.axis_index("c")
        s = lax.axis_index("s")
        wid = c * _NS + s
        sgb, dgb, dsb = (sg0, sg1), (dg0, dg1), (ds0, ds1)
        abufs, bbufs, ubufs = (ab0, ab1), (bb0, bb1), (ub0, ub1)
        ssg, sdg, sds = (ssg0, ssg1), (sdg0, sdg1), (sds0, sds1)
        sas, sbs, sus = (sa0, sa1), (sb0, sb1), (su0, su1)

        for b in range(2):
            pltpu.sync_copy(src_hbm.at[wid, b], sgb[b])
            pltpu.sync_copy(dst_hbm.at[wid, b], dgb[b])
            pltpu.async_copy(t_in_hbm.at[sgb[b]], abufs[b], sas[b])
            pltpu.async_copy(sdst_hbm.at[dgb[b]], bbufs[b], sbs[b])

        def zrow(r, _):
            for j in range(nblk + 1):
                ub0[r, pl.ds(j * 16, 16)] = jnp.zeros((16,), jnp.float32)
            return 0
        lax.fori_loop(0, zc, zrow, 0)
        r0 = s * rpt
        for q in range(rpt // zc):
            pltpu.async_copy(ub0.at[pl.ds(0, zc)],
                             acc.at[pl.ds(r0 + q * zc, zc)], sds0)
        for q in range(rpt // zc):
            pltpu.make_async_copy(ub0.at[pl.ds(0, zc)],
                                  acc.at[pl.ds(r0, zc)], sds0).wait()
        plsc.subcore_barrier()

        lanes = lax.iota(jnp.int32, 16)

        def phase(i, b):
                ab, bb, ub = abufs[b], bbufs[b], ubufs[b]
                pltpu.make_async_copy(t_in_hbm.at[sgb[b]], ab, sas[b]).wait()
                pltpu.make_async_copy(sdst_hbm.at[dgb[b]], bb, sbs[b]).wait()

                @pl.when(i >= 2)
                def _():
                    pltpu.make_async_copy(ub, acc.at[dsb[b]], sus[b]).wait()

                pltpu.async_copy(dst_hbm.at[wid, i], dsb[b], sds[b])

                @pl.when(i + 2 < iters)
                def _():
                    pltpu.async_copy(src_hbm.at[wid, i + 2], sgb[b], ssg[b])
                    pltpu.async_copy(dst_hbm.at[wid, i + 2], dgb[b], sdg[b])

                msk = jnp.int32(-65536)
                for e in range(C):
                    sv = plsc.bitcast(ab[e, pl.ds(feat // 2, 16)],
                                      jnp.float32)
                    ee = sv + bb[e, pl.ds(0, 16)]
                    ee = jnp.where(ee >= 0.0, ee, ee * _ALPHA)
                    ex = jnp.where(lanes < nhead, jnp.exp(ee), 0.0)
                    ub[e, pl.ds(feat, 16)] = ex
                    for g in range(nblk // 2):
                        w32 = ab[e, pl.ds(16 * g, 16)]
                        av = plsc.bitcast(w32 << 16, jnp.float32)
                        bv = plsc.bitcast(w32 & msk, jnp.float32)
                        j0, j1 = 2 * g, 2 * g + 1
                        ub[e, pl.ds(j0 * 16, 16)] = (
                            av * _bcast_lane(ex, j0 * nhead // nblk))
                        ub[e, pl.ds(j1 * 16, 16)] = (
                            bv * _bcast_lane(ex, j1 * nhead // nblk))

                pltpu.make_async_copy(dst_hbm.at[wid, i], dsb[b], sds[b]).wait()
                pltpu.async_copy(ub, acc.at[dsb[b]], sus[b], add=True)

                @pl.when(i + 2 < iters)
                def _():
                    pltpu.make_async_copy(
                        src_hbm.at[wid, i + 2], sgb[b], ssg[b]).wait()
                    pltpu.make_async_copy(
                        dst_hbm.at[wid, i + 2], dgb[b], sdg[b]).wait()
                    pltpu.async_copy(t_in_hbm.at[sgb[b]], ab, sas[b])
                    pltpu.async_copy(sdst_hbm.at[dgb[b]], bb, sbs[b])

        def outer(q, _):
            phase(2 * q, 0)
            phase(2 * q + 1, 1)
            return 0

        lax.fori_loop(0, iters // 2, outer, 0)
        if iters % 2:
            phase(jnp.int32(iters - 1), (iters - 1) % 2)
        pltpu.make_async_copy(ub0, acc.at[ds0], su0).wait()
        pltpu.make_async_copy(ub1, acc.at[ds1], su1).wait()
        plsc.subcore_barrier()
        pltpu.sync_copy(acc.at[pl.ds(s * rpt, rpt)],
                        out_hbm.at[c, pl.ds(s * rpt, rpt)])

    return ek(t_in, sdst_tab, src_rs, dst_rs)


def _pack_bf16_pairs(he, ho):
    be = lax.bitcast_convert_type(he, jnp.int32)
    bo = lax.bitcast_convert_type(ho, jnp.int32)
    rbe = be + 0x7FFF + ((be >> 16) & 1)
    rbo = bo + 0x7FFF + ((bo >> 16) & 1)
    return lax.shift_right_logical(rbe, 16) | (rbo & jnp.int32(-65536))


def _prep1_body(x_ref, w_ref, asrc_ref, adst_ref, pe_ref, po_ref,
                tin_ref, sdst_ref):
    h = jnp.dot(x_ref[...], w_ref[...], preferred_element_type=jnp.float32)
    ssrc = jnp.dot(h, asrc_ref[...], preferred_element_type=jnp.float32)
    sdst = jnp.dot(h, adst_ref[...], preferred_element_type=jnp.float32)
    he = jnp.dot(h, pe_ref[...], preferred_element_type=jnp.float32)
    ho = jnp.dot(h, po_ref[...], preferred_element_type=jnp.float32)
    n = h.shape[0]
    hw = _NFEAT // 2
    tin_ref[:, :hw] = _pack_bf16_pairs(he, ho)
    tin_ref[:, hw:hw + 8] = lax.bitcast_convert_type(ssrc, jnp.int32)
    tin_ref[:, hw + 8:] = jnp.zeros((n, 8), jnp.int32)
    sdst_ref[:, :8] = sdst
    sdst_ref[:, 8:] = jnp.zeros_like(sdst)


def _mid_body(pa_ref, pb_ref, w2_ref, a2_ref, rexp_ref, pe_ref, po_ref,
              tin_ref, sdst_ref):
    t = pa_ref[...] + pb_ref[...]
    u = t[:, :_NFEAT]
    den = jnp.dot(t[:, _NFEAT:_NFEAT + 8], rexp_ref[...],
                  preferred_element_type=jnp.float32)
    h1 = u / (den + 1e-16)
    h1 = jnp.where(h1 > 0.0, h1, jnp.exp(h1) - 1.0)
    g = jnp.dot(h1, w2_ref[...], preferred_element_type=jnp.float32)
    s2 = jnp.dot(g, a2_ref[...], preferred_element_type=jnp.float32)
    ge = jnp.dot(g, pe_ref[...], preferred_element_type=jnp.float32)
    go = jnp.dot(g, po_ref[...], preferred_element_type=jnp.float32)
    n = t.shape[0]
    hw = _NCLASS // 2
    tin_ref[:, :hw] = _pack_bf16_pairs(ge, go)
    tin_ref[:, hw:hw + 1] = lax.bitcast_convert_type(s2[:, 0:1], jnp.int32)
    tin_ref[:, hw + 1:] = jnp.zeros((n, 15), jnp.int32)
    sdst_ref[:, 0:1] = s2[:, 1:2]
    sdst_ref[:, 1:] = jnp.zeros((n, 15), jnp.float32)


def _final_body(pa_ref, pb_ref, out_ref):
    t = pa_ref[...] + pb_ref[...]
    den = t[:, _NCLASS:_NCLASS + 1]
    logits = t[:, :_NCLASS] / (den + 1e-16)
    m = jnp.max(logits, axis=1, keepdims=True)
    lse = m + jnp.log(jnp.sum(jnp.exp(logits - m), axis=1, keepdims=True))
    out_ref[...] = logits - lse


def kernel(x, edge_list, W1, a1, W2, a2):
    epw = _E // (_NC * _NS)
    src_rs1 = edge_list[0].reshape(_NC * _NS, epw // _C1, _C1)
    dst_rs1 = edge_list[1].reshape(_NC * _NS, epw // _C1, _C1)
    src_rs2 = edge_list[0].reshape(_NC * _NS, epw // _C2, _C2)
    dst_rs2 = edge_list[1].reshape(_NC * _NS, epw // _C2, _C2)

    w1all = W1.transpose(1, 0, 2).reshape(_NFEAT, _NHEAD * _NHID)
    eye8 = jnp.eye(_NHEAD, dtype=jnp.float32)
    asrc = (a1[:, :_NHID][:, :, None] * eye8[:, None, :]).reshape(
        _NHEAD * _NHID, _NHEAD)
    adst = (a1[:, _NHID:][:, :, None] * eye8[:, None, :]).reshape(
        _NHEAD * _NHID, _NHEAD)
    rexp = jnp.repeat(eye8, _NHID, axis=1)
    a2m = jnp.stack([a2[0, :_NCLASS], a2[0, _NCLASS:]], axis=1)

    def _pair_sel(n, odd):
        cols = jnp.arange(n // 2)
        srcc = 32 * (cols // 16) + 16 * odd + cols % 16
        return jnp.zeros((n, n // 2), jnp.float32).at[srcc, cols].set(1.0)

    pe1, po1 = _pair_sel(_NFEAT, 0), _pair_sel(_NFEAT, 1)
    pe2, po2 = _pair_sel(_NCLASS, 0), _pair_sel(_NCLASS, 1)

    blk = 1000
    grid = (_N // blk,)

    tin1, sdst1 = pl.pallas_call(
        _prep1_body,
        grid=grid,
        in_specs=[
            pl.BlockSpec((blk, _NFEAT), lambda i: (i, 0)),
            pl.BlockSpec((_NFEAT, _NHEAD * _NHID), lambda i: (0, 0)),
            pl.BlockSpec((_NHEAD * _NHID, _NHEAD), lambda i: (0, 0)),
            pl.BlockSpec((_NHEAD * _NHID, _NHEAD), lambda i: (0, 0)),
            pl.BlockSpec((_NFEAT, _NFEAT // 2), lambda i: (0, 0)),
            pl.BlockSpec((_NFEAT, _NFEAT // 2), lambda i: (0, 0)),
        ],
        out_specs=[
            pl.BlockSpec((blk, _NFEAT // 2 + 16), lambda i: (i, 0)),
            pl.BlockSpec((blk, 16), lambda i: (i, 0)),
        ],
        out_shape=[
            jax.ShapeDtypeStruct((_N, _NFEAT // 2 + 16), jnp.int32),
            jax.ShapeDtypeStruct((_N, 16), jnp.float32),
        ],
    )(x, w1all, asrc, adst, pe1, po1)

    part1 = _edge_pass(tin1, sdst1, src_rs1, dst_rs1, _NFEAT, _NHEAD, _C1)

    tin2, sdst2 = pl.pallas_call(
        _mid_body,
        grid=grid,
        in_specs=[
            pl.BlockSpec((blk, _NFEAT + 16), lambda i: (i, 0)),
            pl.BlockSpec((blk, _NFEAT + 16), lambda i: (i, 0)),
            pl.BlockSpec((_NFEAT, _NCLASS), lambda i: (0, 0)),
            pl.BlockSpec((_NCLASS, 2), lambda i: (0, 0)),
            pl.BlockSpec((_NHEAD, _NFEAT), lambda i: (0, 0)),
            pl.BlockSpec((_NCLASS, _NCLASS // 2), lambda i: (0, 0)),
            pl.BlockSpec((_NCLASS, _NCLASS // 2), lambda i: (0, 0)),
        ],
        out_specs=[
            pl.BlockSpec((blk, _NCLASS // 2 + 16), lambda i: (i, 0)),
            pl.BlockSpec((blk, 16), lambda i: (i, 0)),
        ],
        out_shape=[
            jax.ShapeDtypeStruct((_N, _NCLASS // 2 + 16), jnp.int32),
            jax.ShapeDtypeStruct((_N, 16), jnp.float32),
        ],
    )(part1[0], part1[1], W2[0], a2m, rexp, pe2, po2)

    part2 = _edge_pass(tin2, sdst2, src_rs2, dst_rs2, _NCLASS, 1, _C2)

    out = pl.pallas_call(
        _final_body,
        grid=grid,
        in_specs=[
            pl.BlockSpec((blk, _NCLASS + 16), lambda i: (i, 0)),
            pl.BlockSpec((blk, _NCLASS + 16), lambda i: (i, 0)),
        ],
        out_specs=pl.BlockSpec((blk, _NCLASS), lambda i: (i, 0)),
        out_shape=jax.ShapeDtypeStruct((_N, _NCLASS), jnp.float32),
    )(part2[0], part2[1])

    return out


# speedup vs baseline: 1.0202x; 1.0202x over previous
import functools

import jax
import jax.numpy as jnp
from jax import lax
from jax.experimental import pallas as pl
from jax.experimental.pallas import tpu as pltpu
from jax.experimental.pallas import tpu_sc as plsc

_N = 10000
_E = 320000
_NFEAT = 128
_NHID = 16
_NHEAD = 8
_NCLASS = 32
_ALPHA = 0.2

_NC = 2
_NS = 16
_C1 = 50
_C2 = 125

_GATHER_DNUMS = lax.GatherDimensionNumbers(
    offset_dims=(), collapsed_slice_dims=(0,), start_index_map=(0,))


def _bcast_lane(v, k):
    idx = jnp.full((16, 1), k, dtype=jnp.int32)
    return lax.gather(v, idx, _GATHER_DNUMS, (1,),
                      mode=lax.GatherScatterMode.PROMISE_IN_BOUNDS)


def _edge_pass(t_in, sdst_tab, src_rs, dst_rs, feat, nhead, C):
    row = feat + 16
    nblk = feat // 16
    iters = src_rs.shape[1]
    rpt = _N // _NS
    zc = 125 if C >= 125 else 25
    mesh = plsc.VectorSubcoreMesh(core_axis_name="c", subcore_axis_name="s",
                                  num_cores=_NC, num_subcores=_NS)
    idx_t = pltpu.VMEM((C,), jnp.int32)
    gat_t = pltpu.VMEM((C, feat // 2 + 16), jnp.int32)
    row_t = pltpu.VMEM((C, row), jnp.float32)
    sem_t = pltpu.SemaphoreType.DMA

    @functools.partial(
        pl.kernel,
        out_type=jax.ShapeDtypeStruct((_NC, _N, row), jnp.float32),
        mesh=mesh,
        scratch_types=(
            [pltpu.VMEM_SHARED((_N, row), jnp.float32)]
            + [idx_t] * 6
            + [gat_t, gat_t,
               pltpu.VMEM((C, 16), jnp.float32),
               pltpu.VMEM((C, 16), jnp.float32),
               row_t, row_t]
            + [sem_t] * 12
        ),
        compiler_params=pltpu.CompilerParams(use_tc_tiling_on_sc=False,
                                             needs_layout_passes=False),
    )
    def ek(t_in_hbm, sdst_hbm, src_hbm, dst_hbm, out_hbm,
           acc, sg0, sg1, dg0, dg1, ds0, ds1,
           ab0, ab1, bb0, bb1, ub0, ub1,
           ssg0, ssg1, sdg0, sdg1, sds0, sds1,
           sa0, sa1, sb0, sb1, su0, su1):
        c = lax.axis_index("c")
        s = lax.axis_index("s")
        wid = c * _NS + s
        sgb, dgb, dsb = (sg0, sg1), (dg0, dg1), (ds0, ds1)
        abufs, bbufs, ubufs = (ab0, ab1), (bb0, bb1), (ub0, ub1)
        ssg, sdg, sds = (ssg0, ssg1), (sdg0, sdg1), (sds0, sds1)
        sas, sbs, sus = (sa0, sa1), (sb0, sb1), (su0, su1)

        for b in range(2):
            pltpu.sync_copy(src_hbm.at[wid, b], sgb[b])
            pltpu.sync_copy(dst_hbm.at[wid, b], dgb[b])
            pltpu.async_copy(t_in_hbm.at[sgb[b]], abufs[b], sas[b])
            pltpu.async_copy(sdst_hbm.at[dgb[b]], bbufs[b], sbs[b])

        def zrow(r, _):
            for j in range(nblk + 1):
                ub0[r, pl.ds(j * 16, 16)] = jnp.zeros((16,), jnp.float32)
            return 0
        lax.fori_loop(0, zc, zrow, 0)
        r0 = s * rpt
        for q in range(rpt // zc):
            pltpu.async_copy(ub0.at[pl.ds(0, zc)],
                             acc.at[pl.ds(r0 + q * zc, zc)], sds0)
        for q in range(rpt // zc):
            pltpu.make_async_copy(ub0.at[pl.ds(0, zc)],
                                  acc.at[pl.ds(r0, zc)], sds0).wait()
        plsc.subcore_barrier()

        lanes = lax.iota(jnp.int32, 16)

        def phase(i, b):
                ab, bb, ub = abufs[b], bbufs[b], ubufs[b]
                pltpu.make_async_copy(t_in_hbm.at[sgb[b]], ab, sas[b]).wait()
                pltpu.make_async_copy(sdst_hbm.at[dgb[b]], bb, sbs[b]).wait()

                @pl.when(i >= 2)
                def _():
                    pltpu.make_async_copy(ub, acc.at[dsb[b]], sus[b]).wait()

                pltpu.async_copy(dst_hbm.at[wid, i], dsb[b], sds[b])

                @pl.when(i + 2 < iters)
                def _():
                    pltpu.async_copy(src_hbm.at[wid, i + 2], sgb[b], ssg[b])
                    pltpu.async_copy(dst_hbm.at[wid, i + 2], dgb[b], sdg[b])

                msk = jnp.int32(-65536)
                for e in range(C):
                    sv = plsc.bitcast(ab[e, pl.ds(feat // 2, 16)],
                                      jnp.float32)
                    ee = sv + bb[e, pl.ds(0, 16)]
                    ee = jnp.where(ee >= 0.0, ee, ee * _ALPHA)
                    ex = jnp.where(lanes < nhead, jnp.exp(ee), 0.0)
                    ub[e, pl.ds(feat, 16)] = ex
                    for g in range(nblk // 2):
                        w32 = ab[e, pl.ds(16 * g, 16)]
                        av = plsc.bitcast(w32 << 16, jnp.float32)
                        bv = plsc.bitcast(w32 & msk, jnp.float32)
                        j0, j1 = 2 * g, 2 * g + 1
                        ub[e, pl.ds(j0 * 16, 16)] = (
                            av * _bcast_lane(ex, j0 * nhead // nblk))
                        ub[e, pl.ds(j1 * 16, 16)] = (
                            bv * _bcast_lane(ex, j1 * nhead // nblk))

                pltpu.make_async_copy(dst_hbm.at[wid, i], dsb[b], sds[b]).wait()
                pltpu.async_copy(ub, acc.at[dsb[b]], sus[b], add=True)

                @pl.when(i + 2 < iters)
                def _():
                    pltpu.make_async_copy(
                        src_hbm.at[wid, i + 2], sgb[b], ssg[b]).wait()
                    pltpu.make_async_copy(
                        dst_hbm.at[wid, i + 2], dgb[b], sdg[b]).wait()
                    pltpu.async_copy(t_in_hbm.at[sgb[b]], ab, sas[b])
                    pltpu.async_copy(sdst_hbm.at[dgb[b]], bb, sbs[b])

        def outer(q, _):
            phase(2 * q, 0)
            phase(2 * q + 1, 1)
            return 0

        lax.fori_loop(0, iters // 2, outer, 0)
        if iters % 2:
            phase(jnp.int32(iters - 1), (iters - 1) % 2)
        pltpu.make_async_copy(ub0, acc.at[ds0], su0).wait()
        pltpu.make_async_copy(ub1, acc.at[ds1], su1).wait()
        plsc.subcore_barrier()
        pltpu.sync_copy(acc.at[pl.ds(s * rpt, rpt)],
                        out_hbm.at[c, pl.ds(s * rpt, rpt)])

    return ek(t_in, sdst_tab, src_rs, dst_rs)


def _pack_bf16_pairs(he, ho):
    be = lax.bitcast_convert_type(he, jnp.int32)
    bo = lax.bitcast_convert_type(ho, jnp.int32)
    rbe = be + 0x7FFF + ((be >> 16) & 1)
    rbo = bo + 0x7FFF + ((bo >> 16) & 1)
    return lax.shift_right_logical(rbe, 16) | (rbo & jnp.int32(-65536))


def _prep1_body(x_ref, w_ref, asrc_ref, adst_ref, pe_ref, po_ref,
                tin_ref, sdst_ref):
    h = jnp.dot(x_ref[...], w_ref[...], preferred_element_type=jnp.float32)
    ssrc = jnp.dot(h, asrc_ref[...], preferred_element_type=jnp.float32)
    sdst = jnp.dot(h, adst_ref[...], preferred_element_type=jnp.float32)
    he = jnp.dot(h, pe_ref[...], preferred_element_type=jnp.float32)
    ho = jnp.dot(h, po_ref[...], preferred_element_type=jnp.float32)
    n = h.shape[0]
    hw = _NFEAT // 2
    tin_ref[:, :hw] = _pack_bf16_pairs(he, ho)
    tin_ref[:, hw:hw + 8] = lax.bitcast_convert_type(ssrc, jnp.int32)
    tin_ref[:, hw + 8:] = jnp.zeros((n, 8), jnp.int32)
    sdst_ref[:, :8] = sdst
    sdst_ref[:, 8:] = jnp.zeros_like(sdst)


def _mid_body(pa_ref, pb_ref, w2_ref, a2_ref, rexp_ref, pe_ref, po_ref,
              tin_ref, sdst_ref):
    t = pa_ref[...] + pb_ref[...]
    u = t[:, :_NFEAT]
    den = jnp.dot(t[:, _NFEAT:_NFEAT + 8], rexp_ref[...],
                  preferred_element_type=jnp.float32)
    h1 = u / (den + 1e-16)
    h1 = jnp.where(h1 > 0.0, h1, jnp.exp(h1) - 1.0)
    g = jnp.dot(h1, w2_ref[...], preferred_element_type=jnp.float32)
    s2 = jnp.dot(g, a2_ref[...], preferred_element_type=jnp.float32)
    ge = jnp.dot(g, pe_ref[...], preferred_element_type=jnp.float32)
    go = jnp.dot(g, po_ref[...], preferred_element_type=jnp.float32)
    n = t.shape[0]
    hw = _NCLASS // 2
    tin_ref[:, :hw] = _pack_bf16_pairs(ge, go)
    tin_ref[:, hw:hw + 1] = lax.bitcast_convert_type(s2[:, 0:1], jnp.int32)
    tin_ref[:, hw + 1:] = jnp.zeros((n, 15), jnp.int32)
    sdst_ref[:, 0:1] = s2[:, 1:2]
    sdst_ref[:, 1:] = jnp.zeros((n, 15), jnp.float32)


def _final_body(pa_ref, pb_ref, out_ref):
    t = pa_ref[...] + pb_ref[...]
    den = t[:, _NCLASS:_NCLASS + 1]
    logits = t[:, :_NCLASS] / (den + 1e-16)
    m = jnp.max(logits, axis=1, keepdims=True)
    lse = m + jnp.log(jnp.sum(jnp.exp(logits - m), axis=1, keepdims=True))
    out_ref[...] = logits - lse


def kernel(x, edge_list, W1, a1, W2, a2):
    epw = _E // (_NC * _NS)
    src_rs1 = edge_list[0].reshape(_NC * _NS, epw // _C1, _C1)
    dst_rs1 = edge_list[1].reshape(_NC * _NS, epw // _C1, _C1)
    src_rs2 = edge_list[0].reshape(_NC * _NS, epw // _C2, _C2)
    dst_rs2 = edge_list[1].reshape(_NC * _NS, epw // _C2, _C2)

    w1all = W1.transpose(1, 0, 2).reshape(_NFEAT, _NHEAD * _NHID)
    eye8 = jnp.eye(_NHEAD, dtype=jnp.float32)
    asrc = (a1[:, :_NHID][:, :, None] * eye8[:, None, :]).reshape(
        _NHEAD * _NHID, _NHEAD)
    adst = (a1[:, _NHID:][:, :, None] * eye8[:, None, :]).reshape(
        _NHEAD * _NHID, _NHEAD)
    rexp = jnp.repeat(eye8, _NHID, axis=1)
    a2m = jnp.stack([a2[0, :_NCLASS], a2[0, _NCLASS:]], axis=1)

    def _pair_sel(n, odd):
        cols = jnp.arange(n // 2)
        srcc = 32 * (cols // 16) + 16 * odd + cols % 16
        return jnp.zeros((n, n // 2), jnp.float32).at[srcc, cols].set(1.0)

    pe1, po1 = _pair_sel(_NFEAT, 0), _pair_sel(_NFEAT, 1)
    pe2, po2 = _pair_sel(_NCLASS, 0), _pair_sel(_NCLASS, 1)

    blk = 1000
    grid = (_N // blk,)

    tin1, sdst1 = pl.pallas_call(
        _prep1_body,
        grid=grid,
        in_specs=[
            pl.BlockSpec((blk, _NFEAT), lambda i: (i, 0)),
            pl.BlockSpec((_NFEAT, _NHEAD * _NHID), lambda i: (0, 0)),
            pl.BlockSpec((_NHEAD * _NHID, _NHEAD), lambda i: (0, 0)),
            pl.BlockSpec((_NHEAD * _NHID, _NHEAD), lambda i: (0, 0)),
            pl.BlockSpec((_NFEAT, _NFEAT // 2), lambda i: (0, 0)),
            pl.BlockSpec((_NFEAT, _NFEAT // 2), lambda i: (0, 0)),
        ],
        out_specs=[
            pl.BlockSpec((blk, _NFEAT // 2 + 16), lambda i: (i, 0)),
            pl.BlockSpec((blk, 16), lambda i: (i, 0)),
        ],
        out_shape=[
            jax.ShapeDtypeStruct((_N, _NFEAT // 2 + 16), jnp.int32),
            jax.ShapeDtypeStruct((_N, 16), jnp.float32),
        ],
    )(x, w1all, asrc, adst, pe1, po1)

    part1 = _edge_pass(tin1, sdst1, src_rs1, dst_rs1, _NFEAT, _NHEAD, _C1)

    tin2, sdst2 = pl.pallas_call(
        _mid_body,
        grid=grid,
        in_specs=[
            pl.BlockSpec((blk, _NFEAT + 16), lambda i: (i, 0)),
            pl.BlockSpec((blk, _NFEAT + 16), lambda i: (i, 0)),
            pl.BlockSpec((_NFEAT, _NCLASS), lambda i: (0, 0)),
            pl.BlockSpec((_NCLASS, 2), lambda i: (0, 0)),
            pl.BlockSpec((_NHEAD, _NFEAT), lambda i: (0, 0)),
            pl.BlockSpec((_NCLASS, _NCLASS // 2), lambda i: (0, 0)),
            pl.BlockSpec((_NCLASS, _NCLASS // 2), lambda i: (0, 0)),
        ],
        out_specs=[
            pl.BlockSpec((blk, _NCLASS // 2 + 16), lambda i: (i, 0)),
            pl.BlockSpec((blk, 16), lambda i: (i, 0)),
        ],
        out_shape=[
            jax.ShapeDtypeStruct((_N, _NCLASS // 2 + 16), jnp.int32),
            jax.ShapeDtypeStruct((_N, 16), jnp.float32),
        ],
    )(part1[0], part1[1], W2[0], a2m, rexp, pe2, po2)

    part2 = _edge_pass(tin2, sdst2, src_rs2, dst_rs2, _NCLASS, 1, _C2)

    out = pl.pallas_call(
        _final_body,
        grid=grid,
        in_specs=[
            pl.BlockSpec((blk, _NCLASS + 16), lambda i: (i, 0)),
            pl.BlockSpec((blk, _NCLASS + 16), lambda i: (i, 0)),
        ],
        out_specs=pl.BlockSpec((blk, _NCLASS), lambda i: (i, 0)),
        out_shape=jax.ShapeDtypeStruct((_N, _NCLASS), jnp.float32),
    )(part2[0], part2[1])

    return out


# speedup vs baseline: 1.0417x; 1.0211x over previous
import functools

import jax
import jax.numpy as jnp
from jax import lax
from jax.experimental import pallas as pl
from jax.experimental.pallas import tpu as pltpu
from jax.experimental.pallas import tpu_sc as plsc

_N = 10000
_E = 320000
_NFEAT = 128
_NHID = 16
_NHEAD = 8
_NCLASS = 32
_ALPHA = 0.2

_NC = 2
_NS = 16
_C1 = 50
_C2 = 125

_GATHER_DNUMS = lax.GatherDimensionNumbers(
    offset_dims=(), collapsed_slice_dims=(0,), start_index_map=(0,))


def _bcast_lane(v, k):
    idx = jnp.full((16, 1), k, dtype=jnp.int32)
    return lax.gather(v, idx, _GATHER_DNUMS, (1,),
                      mode=lax.GatherScatterMode.PROMISE_IN_BOUNDS)


def _edge_pass(t_in, sdst_tab, src_rs, dst_rs, feat, nhead, C):
    row = feat + 16
    nblk = feat // 16
    iters = src_rs.shape[1]
    rpt = _N // _NS
    zc = 125 if C >= 125 else 25
    mesh = plsc.VectorSubcoreMesh(core_axis_name="c", subcore_axis_name="s",
                                  num_cores=_NC, num_subcores=_NS)
    idx_t = pltpu.VMEM((C,), jnp.int32)
    gat_t = pltpu.VMEM((C, feat // 2 + 16), jnp.int32)
    row_t = pltpu.VMEM((C, row), jnp.float32)
    sem_t = pltpu.SemaphoreType.DMA

    @functools.partial(
        pl.kernel,
        out_type=jax.ShapeDtypeStruct((_NC, _N, row), jnp.float32),
        mesh=mesh,
        scratch_types=(
            [pltpu.VMEM_SHARED((_N, row), jnp.float32)]
            + [idx_t] * 6
            + [gat_t, gat_t,
               pltpu.VMEM((C, 16), jnp.float32),
               pltpu.VMEM((C, 16), jnp.float32),
               row_t, row_t]
            + [sem_t] * 12
        ),
        compiler_params=pltpu.CompilerParams(use_tc_tiling_on_sc=False,
                                             needs_layout_passes=False),
    )
    def ek(t_in_hbm, sdst_hbm, src_hbm, dst_hbm, out_hbm,
           acc, sg0, sg1, dg0, dg1, ds0, ds1,
           ab0, ab1, bb0, bb1, ub0, ub1,
           ssg0, ssg1, sdg0, sdg1, sds0, sds1,
           sa0, sa1, sb0, sb1, su0, su1):
        c = lax.axis_index("c")
        s = lax.axis_index("s")
        wid = c * _NS + s
        sgb, dgb, dsb = (sg0, sg1), (dg0, dg1), (ds0, ds1)
        abufs, bbufs, ubufs = (ab0, ab1), (bb0, bb1), (ub0, ub1)
        ssg, sdg, sds = (ssg0, ssg1), (sdg0, sdg1), (sds0, sds1)
        sas, sbs, sus = (sa0, sa1), (sb0, sb1), (su0, su1)

        for b in range(2):
            pltpu.sync_copy(src_hbm.at[wid, b], sgb[b])
            pltpu.sync_copy(dst_hbm.at[wid, b], dgb[b])
            pltpu.async_copy(t_in_hbm.at[sgb[b]], abufs[b], sas[b])
            pltpu.async_copy(sdst_hbm.at[dgb[b]], bbufs[b], sbs[b])

        def zrow(r, _):
            for j in range(nblk + 1):
                ub0[r, pl.ds(j * 16, 16)] = jnp.zeros((16,), jnp.float32)
            return 0
        lax.fori_loop(0, zc, zrow, 0)
        r0 = s * rpt
        for q in range(rpt // zc):
            pltpu.async_copy(ub0.at[pl.ds(0, zc)],
                             acc.at[pl.ds(r0 + q * zc, zc)], sds0)
        for q in range(rpt // zc):
            pltpu.make_async_copy(ub0.at[pl.ds(0, zc)],
                                  acc.at[pl.ds(r0, zc)], sds0).wait()
        plsc.subcore_barrier()

        lanes = lax.iota(jnp.int32, 16)

        def phase(i, b):
                ab, bb, ub = abufs[b], bbufs[b], ubufs[b]
                pltpu.make_async_copy(t_in_hbm.at[sgb[b]], ab, sas[b]).wait()
                pltpu.make_async_copy(sdst_hbm.at[dgb[b]], bb, sbs[b]).wait()

                @pl.when(i >= 2)
                def _():
                    pltpu.make_async_copy(ub, acc.at[dsb[b]], sus[b]).wait()

                pltpu.async_copy(dst_hbm.at[wid, i], dsb[b], sds[b])

                @pl.when(i + 2 < iters)
                def _():
                    pltpu.async_copy(src_hbm.at[wid, i + 2], sgb[b], ssg[b])
                    pltpu.async_copy(dst_hbm.at[wid, i + 2], dgb[b], sdg[b])

                msk = jnp.int32(-65536)
                for e in range(C):
                    sv = plsc.bitcast(ab[e, pl.ds(feat // 2, 16)],
                                      jnp.float32)
                    ee = sv + bb[e, pl.ds(0, 16)]
                    ee = jnp.where(ee >= 0.0, ee, ee * _ALPHA)
                    ex = jnp.where(lanes < nhead, jnp.exp(ee), 0.0)
                    ub[e, pl.ds(feat, 16)] = ex
                    for g in range(nblk // 2):
                        w32 = ab[e, pl.ds(16 * g, 16)]
                        av = plsc.bitcast(w32 << 16, jnp.float32)
                        bv = plsc.bitcast(w32 & msk, jnp.float32)
                        j0, j1 = 2 * g, 2 * g + 1
                        ub[e, pl.ds(j0 * 16, 16)] = (
                            av * _bcast_lane(ex, j0 * nhead // nblk))
                        ub[e, pl.ds(j1 * 16, 16)] = (
                            bv * _bcast_lane(ex, j1 * nhead // nblk))

                pltpu.make_async_copy(dst_hbm.at[wid, i], dsb[b], sds[b]).wait()
                pltpu.async_copy(ub, acc.at[dsb[b]], sus[b], add=True)

                @pl.when(i + 2 < iters)
                def _():
                    pltpu.make_async_copy(
                        src_hbm.at[wid, i + 2], sgb[b], ssg[b]).wait()
                    pltpu.make_async_copy(
                        dst_hbm.at[wid, i + 2], dgb[b], sdg[b]).wait()
                    pltpu.async_copy(t_in_hbm.at[sgb[b]], ab, sas[b])
                    pltpu.async_copy(sdst_hbm.at[dgb[b]], bb, sbs[b])

        def outer(q, _):
            phase(2 * q, 0)
            phase(2 * q + 1, 1)
            return 0

        lax.fori_loop(0, iters // 2, outer, 0)
        if iters % 2:
            phase(jnp.int32(iters - 1), (iters - 1) % 2)
        pltpu.make_async_copy(ub0, acc.at[ds0], su0).wait()
        pltpu.make_async_copy(ub1, acc.at[ds1], su1).wait()
        plsc.subcore_barrier()
        pltpu.sync_copy(acc.at[pl.ds(s * rpt, rpt)],
                        out_hbm.at[c, pl.ds(s * rpt, rpt)])

    return ek(t_in, sdst_tab, src_rs, dst_rs)


def _pack_bf16_pairs(he, ho):
    be = lax.bitcast_convert_type(he, jnp.int32)
    bo = lax.bitcast_convert_type(ho, jnp.int32)
    rbe = be + 0x7FFF + ((be >> 16) & 1)
    rbo = bo + 0x7FFF + ((bo >> 16) & 1)
    return lax.shift_right_logical(rbe, 16) | (rbo & jnp.int32(-65536))


def _prep1_body(x_ref, w_ref, asrc_ref, adst_ref, pe_ref, po_ref,
                tin_ref, sdst_ref):
    h = jnp.dot(x_ref[...], w_ref[...], preferred_element_type=jnp.float32)
    ssrc = jnp.dot(h, asrc_ref[...], preferred_element_type=jnp.float32)
    sdst = jnp.dot(h, adst_ref[...], preferred_element_type=jnp.float32)
    he = jnp.dot(h, pe_ref[...], preferred_element_type=jnp.float32)
    ho = jnp.dot(h, po_ref[...], preferred_element_type=jnp.float32)
    n = h.shape[0]
    hw = _NFEAT // 2
    tin_ref[:, :hw] = _pack_bf16_pairs(he, ho)
    tin_ref[:, hw:hw + 8] = lax.bitcast_convert_type(ssrc, jnp.int32)
    tin_ref[:, hw + 8:] = jnp.zeros((n, 8), jnp.int32)
    sdst_ref[:, :8] = sdst
    sdst_ref[:, 8:] = jnp.zeros_like(sdst)


def _mid_body(pa_ref, pb_ref, w2_ref, a2_ref, rexp_ref, pe_ref, po_ref,
              tin_ref, sdst_ref):
    t = pa_ref[...] + pb_ref[...]
    u = t[:, :_NFEAT]
    den = jnp.dot(t[:, _NFEAT:_NFEAT + 8], rexp_ref[...],
                  preferred_element_type=jnp.float32)
    h1 = u / (den + 1e-16)
    h1 = jnp.where(h1 > 0.0, h1, jnp.exp(h1) - 1.0)
    g = jnp.dot(h1, w2_ref[...], preferred_element_type=jnp.float32)
    s2 = jnp.dot(g, a2_ref[...], preferred_element_type=jnp.float32)
    ge = jnp.dot(g, pe_ref[...], preferred_element_type=jnp.float32)
    go = jnp.dot(g, po_ref[...], preferred_element_type=jnp.float32)
    n = t.shape[0]
    hw = _NCLASS // 2
    tin_ref[:, :hw] = _pack_bf16_pairs(ge, go)
    tin_ref[:, hw:hw + 1] = lax.bitcast_convert_type(s2[:, 0:1], jnp.int32)
    tin_ref[:, hw + 1:] = jnp.zeros((n, 15), jnp.int32)
    sdst_ref[:, 0:1] = s2[:, 1:2]
    sdst_ref[:, 1:] = jnp.zeros((n, 15), jnp.float32)


def _final_body(pa_ref, pb_ref, out_ref):
    t = pa_ref[...] + pb_ref[...]
    den = t[:, _NCLASS:_NCLASS + 1]
    logits = t[:, :_NCLASS] / (den + 1e-16)
    m = jnp.max(logits, axis=1, keepdims=True)
    lse = m + jnp.log(jnp.sum(jnp.exp(logits - m), axis=1, keepdims=True))
    out_ref[...] = logits - lse


def kernel(x, edge_list, W1, a1, W2, a2):
    epw = _E // (_NC * _NS)
    src_rs1 = edge_list[0].reshape(_NC * _NS, epw // _C1, _C1)
    dst_rs1 = edge_list[1].reshape(_NC * _NS, epw // _C1, _C1)
    src_rs2 = edge_list[0].reshape(_NC * _NS, epw // _C2, _C2)
    dst_rs2 = edge_list[1].reshape(_NC * _NS, epw // _C2, _C2)

    w1all = W1.transpose(1, 0, 2).reshape(_NFEAT, _NHEAD * _NHID)
    eye8 = jnp.eye(_NHEAD, dtype=jnp.float32)
    asrc = (a1[:, :_NHID][:, :, None] * eye8[:, None, :]).reshape(
        _NHEAD * _NHID, _NHEAD)
    adst = (a1[:, _NHID:][:, :, None] * eye8[:, None, :]).reshape(
        _NHEAD * _NHID, _NHEAD)
    rexp = jnp.repeat(eye8, _NHID, axis=1)
    a2m = jnp.stack([a2[0, :_NCLASS], a2[0, _NCLASS:]], axis=1)

    def _pair_sel(n, odd):
        cols = jnp.arange(n // 2)
        srcc = 32 * (cols // 16) + 16 * odd + cols % 16
        return (jnp.arange(n)[:, None] == srcc[None, :]).astype(jnp.float32)

    pe1, po1 = _pair_sel(_NFEAT, 0), _pair_sel(_NFEAT, 1)
    pe2, po2 = _pair_sel(_NCLASS, 0), _pair_sel(_NCLASS, 1)

    blk = 1000
    grid = (_N // blk,)

    tin1, sdst1 = pl.pallas_call(
        _prep1_body,
        grid=grid,
        in_specs=[
            pl.BlockSpec((blk, _NFEAT), lambda i: (i, 0)),
            pl.BlockSpec((_NFEAT, _NHEAD * _NHID), lambda i: (0, 0)),
            pl.BlockSpec((_NHEAD * _NHID, _NHEAD), lambda i: (0, 0)),
            pl.BlockSpec((_NHEAD * _NHID, _NHEAD), lambda i: (0, 0)),
            pl.BlockSpec((_NFEAT, _NFEAT // 2), lambda i: (0, 0)),
            pl.BlockSpec((_NFEAT, _NFEAT // 2), lambda i: (0, 0)),
        ],
        out_specs=[
            pl.BlockSpec((blk, _NFEAT // 2 + 16), lambda i: (i, 0)),
            pl.BlockSpec((blk, 16), lambda i: (i, 0)),
        ],
        out_shape=[
            jax.ShapeDtypeStruct((_N, _NFEAT // 2 + 16), jnp.int32),
            jax.ShapeDtypeStruct((_N, 16), jnp.float32),
        ],
    )(x, w1all, asrc, adst, pe1, po1)

    part1 = _edge_pass(tin1, sdst1, src_rs1, dst_rs1, _NFEAT, _NHEAD, _C1)

    tin2, sdst2 = pl.pallas_call(
        _mid_body,
        grid=grid,
        in_specs=[
            pl.BlockSpec((blk, _NFEAT + 16), lambda i: (i, 0)),
            pl.BlockSpec((blk, _NFEAT + 16), lambda i: (i, 0)),
            pl.BlockSpec((_NFEAT, _NCLASS), lambda i: (0, 0)),
            pl.BlockSpec((_NCLASS, 2), lambda i: (0, 0)),
            pl.BlockSpec((_NHEAD, _NFEAT), lambda i: (0, 0)),
            pl.BlockSpec((_NCLASS, _NCLASS // 2), lambda i: (0, 0)),
            pl.BlockSpec((_NCLASS, _NCLASS // 2), lambda i: (0, 0)),
        ],
        out_specs=[
            pl.BlockSpec((blk, _NCLASS // 2 + 16), lambda i: (i, 0)),
            pl.BlockSpec((blk, 16), lambda i: (i, 0)),
        ],
        out_shape=[
            jax.ShapeDtypeStruct((_N, _NCLASS // 2 + 16), jnp.int32),
            jax.ShapeDtypeStruct((_N, 16), jnp.float32),
        ],
    )(part1[0], part1[1], W2[0], a2m, rexp, pe2, po2)

    part2 = _edge_pass(tin2, sdst2, src_rs2, dst_rs2, _NCLASS, 1, _C2)

    out = pl.pallas_call(
        _final_body,
        grid=grid,
        in_specs=[
            pl.BlockSpec((blk, _NCLASS + 16), lambda i: (i, 0)),
            pl.BlockSpec((blk, _NCLASS + 16), lambda i: (i, 0)),
        ],
        out_specs=pl.BlockSpec((blk, _NCLASS), lambda i: (i, 0)),
        out_shape=jax.ShapeDtypeStruct((_N, _NCLASS), jnp.float32),
    )(part2[0], part2[1])

    return out


# speedup vs baseline: 1.0914x; 1.0476x over previous
import functools

import jax
import jax.numpy as jnp
from jax import lax
from jax.experimental import pallas as pl
from jax.experimental.pallas import tpu as pltpu
from jax.experimental.pallas import tpu_sc as plsc

_N = 10000
_E = 320000
_NFEAT = 128
_NHID = 16
_NHEAD = 8
_NCLASS = 32
_ALPHA = 0.2

_NC = 2
_NS = 16
_C1 = 50
_C2 = 125

_GATHER_DNUMS = lax.GatherDimensionNumbers(
    offset_dims=(), collapsed_slice_dims=(0,), start_index_map=(0,))


def _bcast_lane(v, k):
    idx = jnp.full((16, 1), k, dtype=jnp.int32)
    return lax.gather(v, idx, _GATHER_DNUMS, (1,),
                      mode=lax.GatherScatterMode.PROMISE_IN_BOUNDS)


def _edge_pass(t_in, sdst_tab, src_rs, dst_rs, feat, nhead, C):
    row = feat + 16
    nblk = feat // 16
    iters = src_rs.shape[1]
    rpt = _N // _NS
    zc = 125 if C >= 125 else 25
    mesh = plsc.VectorSubcoreMesh(core_axis_name="c", subcore_axis_name="s",
                                  num_cores=_NC, num_subcores=_NS)
    idx_t = pltpu.VMEM((C,), jnp.int32)
    gat_t = pltpu.VMEM((C, feat // 2 + 16), jnp.int32)
    row_t = pltpu.VMEM((C, row), jnp.float32)
    sem_t = pltpu.SemaphoreType.DMA

    @functools.partial(
        pl.kernel,
        out_type=[jax.ShapeDtypeStruct((_N, row), jnp.float32),
                  jax.ShapeDtypeStruct((_N, row), jnp.float32)],
        mesh=mesh,
        scratch_types=(
            [pltpu.VMEM_SHARED((_N, row), jnp.float32)]
            + [idx_t] * 6
            + [gat_t, gat_t,
               pltpu.VMEM((C, 16), jnp.float32),
               pltpu.VMEM((C, 16), jnp.float32),
               row_t, row_t]
            + [sem_t] * 12
        ),
        compiler_params=pltpu.CompilerParams(use_tc_tiling_on_sc=False,
                                             needs_layout_passes=False),
    )
    def ek(t_in_hbm, sdst_hbm, src_hbm, dst_hbm, out0_hbm, out1_hbm,
           acc, sg0, sg1, dg0, dg1, ds0, ds1,
           ab0, ab1, bb0, bb1, ub0, ub1,
           ssg0, ssg1, sdg0, sdg1, sds0, sds1,
           sa0, sa1, sb0, sb1, su0, su1):
        c = lax.axis_index("c")
        s = lax.axis_index("s")
        wid = c * _NS + s
        sgb, dgb, dsb = (sg0, sg1), (dg0, dg1), (ds0, ds1)
        abufs, bbufs, ubufs = (ab0, ab1), (bb0, bb1), (ub0, ub1)
        ssg, sdg, sds = (ssg0, ssg1), (sdg0, sdg1), (sds0, sds1)
        sas, sbs, sus = (sa0, sa1), (sb0, sb1), (su0, su1)

        for b in range(2):
            pltpu.sync_copy(src_hbm.at[wid, b], sgb[b])
            pltpu.sync_copy(dst_hbm.at[wid, b], dgb[b])
            pltpu.async_copy(t_in_hbm.at[sgb[b]], abufs[b], sas[b])
            pltpu.async_copy(sdst_hbm.at[dgb[b]], bbufs[b], sbs[b])

        def zrow(r, _):
            for j in range(nblk + 1):
                ub0[r, pl.ds(j * 16, 16)] = jnp.zeros((16,), jnp.float32)
            return 0
        lax.fori_loop(0, zc, zrow, 0)
        r0 = s * rpt
        for q in range(rpt // zc):
            pltpu.async_copy(ub0.at[pl.ds(0, zc)],
                             acc.at[pl.ds(r0 + q * zc, zc)], sds0)
        for q in range(rpt // zc):
            pltpu.make_async_copy(ub0.at[pl.ds(0, zc)],
                                  acc.at[pl.ds(r0, zc)], sds0).wait()
        plsc.subcore_barrier()

        lanes = lax.iota(jnp.int32, 16)

        def phase(i, b):
                ab, bb, ub = abufs[b], bbufs[b], ubufs[b]
                pltpu.make_async_copy(t_in_hbm.at[sgb[b]], ab, sas[b]).wait()
                pltpu.make_async_copy(sdst_hbm.at[dgb[b]], bb, sbs[b]).wait()

                @pl.when(i >= 2)
                def _():
                    pltpu.make_async_copy(ub, acc.at[dsb[b]], sus[b]).wait()

                pltpu.async_copy(dst_hbm.at[wid, i], dsb[b], sds[b])

                @pl.when(i + 2 < iters)
                def _():
                    pltpu.async_copy(src_hbm.at[wid, i + 2], sgb[b], ssg[b])
                    pltpu.async_copy(dst_hbm.at[wid, i + 2], dgb[b], sdg[b])

                msk = jnp.int32(-65536)
                for e in range(C):
                    sv = plsc.bitcast(ab[e, pl.ds(feat // 2, 16)],
                                      jnp.float32)
                    ee = sv + bb[e, pl.ds(0, 16)]
                    ee = jnp.where(ee >= 0.0, ee, ee * _ALPHA)
                    ex = jnp.where(lanes < nhead, jnp.exp(ee), 0.0)
                    ub[e, pl.ds(feat, 16)] = ex
                    for g in range(nblk // 2):
                        w32 = ab[e, pl.ds(16 * g, 16)]
                        av = plsc.bitcast(w32 << 16, jnp.float32)
                        bv = plsc.bitcast(w32 & msk, jnp.float32)
                        j0, j1 = 2 * g, 2 * g + 1
                        ub[e, pl.ds(j0 * 16, 16)] = (
                            av * _bcast_lane(ex, j0 * nhead // nblk))
                        ub[e, pl.ds(j1 * 16, 16)] = (
                            bv * _bcast_lane(ex, j1 * nhead // nblk))

                pltpu.make_async_copy(dst_hbm.at[wid, i], dsb[b], sds[b]).wait()
                pltpu.async_copy(ub, acc.at[dsb[b]], sus[b], add=True)

                @pl.when(i + 2 < iters)
                def _():
                    pltpu.make_async_copy(
                        src_hbm.at[wid, i + 2], sgb[b], ssg[b]).wait()
                    pltpu.make_async_copy(
                        dst_hbm.at[wid, i + 2], dgb[b], sdg[b]).wait()
                    pltpu.async_copy(t_in_hbm.at[sgb[b]], ab, sas[b])
                    pltpu.async_copy(sdst_hbm.at[dgb[b]], bb, sbs[b])

        def outer(q, _):
            phase(2 * q, 0)
            phase(2 * q + 1, 1)
            return 0

        lax.fori_loop(0, iters // 2, outer, 0)
        if iters % 2:
            phase(jnp.int32(iters - 1), (iters - 1) % 2)
        pltpu.make_async_copy(ub0, acc.at[ds0], su0).wait()
        pltpu.make_async_copy(ub1, acc.at[ds1], su1).wait()
        plsc.subcore_barrier()

        @pl.when(c == 0)
        def _():
            pltpu.sync_copy(acc.at[pl.ds(s * rpt, rpt)],
                            out0_hbm.at[pl.ds(s * rpt, rpt)])

        @pl.when(c == 1)
        def _():
            pltpu.sync_copy(acc.at[pl.ds(s * rpt, rpt)],
                            out1_hbm.at[pl.ds(s * rpt, rpt)])

    return ek(t_in, sdst_tab, src_rs, dst_rs)


def _pack_bf16_pairs(he, ho):
    be = lax.bitcast_convert_type(he, jnp.int32)
    bo = lax.bitcast_convert_type(ho, jnp.int32)
    rbe = be + 0x7FFF + ((be >> 16) & 1)
    rbo = bo + 0x7FFF + ((bo >> 16) & 1)
    return lax.shift_right_logical(rbe, 16) | (rbo & jnp.int32(-65536))


def _prep1_body(x_ref, w_ref, asrc_ref, adst_ref, pe_ref, po_ref,
                tin_ref, sdst_ref):
    h = jnp.dot(x_ref[...], w_ref[...], preferred_element_type=jnp.float32)
    ssrc = jnp.dot(h, asrc_ref[...], preferred_element_type=jnp.float32)
    sdst = jnp.dot(h, adst_ref[...], preferred_element_type=jnp.float32)
    he = jnp.dot(h, pe_ref[...], preferred_element_type=jnp.float32)
    ho = jnp.dot(h, po_ref[...], preferred_element_type=jnp.float32)
    n = h.shape[0]
    hw = _NFEAT // 2
    tin_ref[:, :hw] = _pack_bf16_pairs(he, ho)
    tin_ref[:, hw:hw + 8] = lax.bitcast_convert_type(ssrc, jnp.int32)
    tin_ref[:, hw + 8:] = jnp.zeros((n, 8), jnp.int32)
    sdst_ref[:, :8] = sdst
    sdst_ref[:, 8:] = jnp.zeros_like(sdst)


def _mid_body(pa_ref, pb_ref, w2_ref, a2_ref, rexp_ref, pe_ref, po_ref,
              tin_ref, sdst_ref):
    t = pa_ref[...] + pb_ref[...]
    u = t[:, :_NFEAT]
    den = jnp.dot(t[:, _NFEAT:_NFEAT + 8], rexp_ref[...],
                  preferred_element_type=jnp.float32)
    h1 = u / (den + 1e-16)
    h1 = jnp.where(h1 > 0.0, h1, jnp.exp(h1) - 1.0)
    g = jnp.dot(h1, w2_ref[...], preferred_element_type=jnp.float32)
    s2 = jnp.dot(g, a2_ref[...], preferred_element_type=jnp.float32)
    ge = jnp.dot(g, pe_ref[...], preferred_element_type=jnp.float32)
    go = jnp.dot(g, po_ref[...], preferred_element_type=jnp.float32)
    n = t.shape[0]
    hw = _NCLASS // 2
    tin_ref[:, :hw] = _pack_bf16_pairs(ge, go)
    tin_ref[:, hw:hw + 1] = lax.bitcast_convert_type(s2[:, 0:1], jnp.int32)
    tin_ref[:, hw + 1:] = jnp.zeros((n, 15), jnp.int32)
    sdst_ref[:, 0:1] = s2[:, 1:2]
    sdst_ref[:, 1:] = jnp.zeros((n, 15), jnp.float32)


def _final_body(pa_ref, pb_ref, out_ref):
    t = pa_ref[...] + pb_ref[...]
    den = t[:, _NCLASS:_NCLASS + 1]
    logits = t[:, :_NCLASS] / (den + 1e-16)
    m = jnp.max(logits, axis=1, keepdims=True)
    lse = m + jnp.log(jnp.sum(jnp.exp(logits - m), axis=1, keepdims=True))
    out_ref[...] = logits - lse


def kernel(x, edge_list, W1, a1, W2, a2):
    epw = _E // (_NC * _NS)
    src_rs1 = edge_list[0].reshape(_NC * _NS, epw // _C1, _C1)
    dst_rs1 = edge_list[1].reshape(_NC * _NS, epw // _C1, _C1)
    src_rs2 = edge_list[0].reshape(_NC * _NS, epw // _C2, _C2)
    dst_rs2 = edge_list[1].reshape(_NC * _NS, epw // _C2, _C2)

    w1all = W1.transpose(1, 0, 2).reshape(_NFEAT, _NHEAD * _NHID)
    eye8 = jnp.eye(_NHEAD, dtype=jnp.float32)
    asrc = (a1[:, :_NHID][:, :, None] * eye8[:, None, :]).reshape(
        _NHEAD * _NHID, _NHEAD)
    adst = (a1[:, _NHID:][:, :, None] * eye8[:, None, :]).reshape(
        _NHEAD * _NHID, _NHEAD)
    rexp = jnp.repeat(eye8, _NHID, axis=1)
    a2m = jnp.stack([a2[0, :_NCLASS], a2[0, _NCLASS:]], axis=1)

    def _pair_sel(n, odd):
        cols = jnp.arange(n // 2)
        srcc = 32 * (cols // 16) + 16 * odd + cols % 16
        return (jnp.arange(n)[:, None] == srcc[None, :]).astype(jnp.float32)

    pe1, po1 = _pair_sel(_NFEAT, 0), _pair_sel(_NFEAT, 1)
    pe2, po2 = _pair_sel(_NCLASS, 0), _pair_sel(_NCLASS, 1)

    blk = 1000
    grid = (_N // blk,)

    tin1, sdst1 = pl.pallas_call(
        _prep1_body,
        grid=grid,
        in_specs=[
            pl.BlockSpec((blk, _NFEAT), lambda i: (i, 0)),
            pl.BlockSpec((_NFEAT, _NHEAD * _NHID), lambda i: (0, 0)),
            pl.BlockSpec((_NHEAD * _NHID, _NHEAD), lambda i: (0, 0)),
            pl.BlockSpec((_NHEAD * _NHID, _NHEAD), lambda i: (0, 0)),
            pl.BlockSpec((_NFEAT, _NFEAT // 2), lambda i: (0, 0)),
            pl.BlockSpec((_NFEAT, _NFEAT // 2), lambda i: (0, 0)),
        ],
        out_specs=[
            pl.BlockSpec((blk, _NFEAT // 2 + 16), lambda i: (i, 0)),
            pl.BlockSpec((blk, 16), lambda i: (i, 0)),
        ],
        out_shape=[
            jax.ShapeDtypeStruct((_N, _NFEAT // 2 + 16), jnp.int32),
            jax.ShapeDtypeStruct((_N, 16), jnp.float32),
        ],
    )(x, w1all, asrc, adst, pe1, po1)

    part1 = _edge_pass(tin1, sdst1, src_rs1, dst_rs1, _NFEAT, _NHEAD, _C1)

    tin2, sdst2 = pl.pallas_call(
        _mid_body,
        grid=grid,
        in_specs=[
            pl.BlockSpec((blk, _NFEAT + 16), lambda i: (i, 0)),
            pl.BlockSpec((blk, _NFEAT + 16), lambda i: (i, 0)),
            pl.BlockSpec((_NFEAT, _NCLASS), lambda i: (0, 0)),
            pl.BlockSpec((_NCLASS, 2), lambda i: (0, 0)),
            pl.BlockSpec((_NHEAD, _NFEAT), lambda i: (0, 0)),
            pl.BlockSpec((_NCLASS, _NCLASS // 2), lambda i: (0, 0)),
            pl.BlockSpec((_NCLASS, _NCLASS // 2), lambda i: (0, 0)),
        ],
        out_specs=[
            pl.BlockSpec((blk, _NCLASS // 2 + 16), lambda i: (i, 0)),
            pl.BlockSpec((blk, 16), lambda i: (i, 0)),
        ],
        out_shape=[
            jax.ShapeDtypeStruct((_N, _NCLASS // 2 + 16), jnp.int32),
            jax.ShapeDtypeStruct((_N, 16), jnp.float32),
        ],
    )(part1[0], part1[1], W2[0], a2m, rexp, pe2, po2)

    part2 = _edge_pass(tin2, sdst2, src_rs2, dst_rs2, _NCLASS, 1, _C2)

    out = pl.pallas_call(
        _final_body,
        grid=grid,
        in_specs=[
            pl.BlockSpec((blk, _NCLASS + 16), lambda i: (i, 0)),
            pl.BlockSpec((blk, _NCLASS + 16), lambda i: (i, 0)),
        ],
        out_specs=pl.BlockSpec((blk, _NCLASS), lambda i: (i, 0)),
        out_shape=jax.ShapeDtypeStruct((_N, _NCLASS), jnp.float32),
    )(part2[0], part2[1])

    return out
